# baseline (speedup 1.0000x reference)
import functools

import jax
import jax.numpy as jnp
from jax import lax
from jax.experimental import pallas as pl
from jax.experimental.pallas import tpu as pltpu
from jax.experimental.pallas import tpu_sc as plsc

_VOCAB = 1000
_EMBED = 20
_OUT = 10
_BATCH = 16384
_PAD = 16
_NC, _NS = 2, 16
_NW = _NC * _NS
_PER_W = _BATCH // _NW
_CHUNK = 128
_NCHUNK = _PER_W // _CHUNK


def _build_table_body(t_ref, w_ref, b_ref, o_ref):
    t = t_ref[...]
    w = w_ref[...]
    acc = jnp.broadcast_to(b_ref[...], (_VOCAB, _PAD))
    for k in range(_EMBED):
        acc = acc + t[:, k:k + 1] * w[k:k + 1, :]
    o_ref[...] = acc


def _gather_body(p_hbm, i1_hbm, i2_hbm, y1_hbm, y2_hbm,
                 i1_v, i2_v, r1_v, r2_v, sem, osem):
    wid = lax.axis_index("s") * _NC + lax.axis_index("c")
    base = wid * _PER_W
    pltpu.sync_copy(i1_hbm.at[wid], i1_v)
    pltpu.sync_copy(i2_hbm.at[wid], i2_v)
    copies = []
    for j in range(_NCHUNK):
        copies.append(pltpu.async_copy(
            p_hbm.at[i1_v.at[j]],
            r1_v.at[pl.ds(j * _CHUNK, _CHUNK)], sem))
        copies.append(pltpu.async_copy(
            p_hbm.at[i2_v.at[j]],
            r2_v.at[pl.ds(j * _CHUNK, _CHUNK)], sem))
    for c in copies:
        c.wait()
    out1 = pltpu.async_copy(r1_v, y1_hbm.at[pl.ds(base, _PER_W)], osem)
    out2 = pltpu.async_copy(r2_v, y2_hbm.at[pl.ds(base, _PER_W)], osem)
    out1.wait()
    out2.wait()


def _sc_gather(p, idx1r, idx2r):
    mesh = plsc.VectorSubcoreMesh(core_axis_name="c", subcore_axis_name="s")
    run = pl.kernel(
        _gather_body,
        out_type=(
            jax.ShapeDtypeStruct((_BATCH, _PAD), jnp.float32),
            jax.ShapeDtypeStruct((_BATCH, _PAD), jnp.float32),
        ),
        mesh=mesh,
        compiler_params=pltpu.CompilerParams(use_tc_tiling_on_sc=False),
        scratch_types=[
            pltpu.VMEM((_NCHUNK, _CHUNK), jnp.int32),
            pltpu.VMEM((_NCHUNK, _CHUNK), jnp.int32),
            pltpu.VMEM((_PER_W, _PAD), jnp.float32),
            pltpu.VMEM((_PER_W, _PAD), jnp.float32),
            pltpu.SemaphoreType.DMA,
            pltpu.SemaphoreType.DMA,
        ],
    )
    return run(p, idx1r, idx2r)


def _finish_body(y1_ref, y2_ref, loss_ref):
    ap = y1_ref[...]
    cp = y2_ref[...]
    num128 = jnp.sum(ap * cp, axis=0)[None, :]
    s1_128 = jnp.sum(ap * ap, axis=0)[None, :]
    s2_128 = jnp.sum(cp * cp, axis=0)[None, :]
    num = jnp.zeros((1, _PAD), jnp.float32)
    s1 = jnp.zeros((1, _PAD), jnp.float32)
    s2 = jnp.zeros((1, _PAD), jnp.float32)
    for j in range(8):
        num = num + num128[:, j * _PAD:(j + 1) * _PAD]
        s1 = s1 + s1_128[:, j * _PAD:(j + 1) * _PAD]
        s2 = s2 + s2_128[:, j * _PAD:(j + 1) * _PAD]
    denom = jnp.maximum(jnp.sqrt(s1) * jnp.sqrt(s2), 1e-8)
    loss_ref[...] = (num / denom)[:, :_OUT]


def kernel(DPTD_name_1, DPTD_name_2, table, W, b):
    wt = jnp.zeros((_EMBED, _PAD), jnp.float32).at[:, :_OUT].set(W.T)
    bp = jnp.zeros((1, _PAD), jnp.float32).at[:, :_OUT].set(b[None, :])
    p = pl.pallas_call(
        _build_table_body,
        out_shape=jax.ShapeDtypeStruct((_VOCAB, _PAD), jnp.float32),
    )(table, wt, bp)
    idx1r = DPTD_name_1.astype(jnp.int32).reshape(_NW, _NCHUNK, _CHUNK)
    idx2r = DPTD_name_2.astype(jnp.int32).reshape(_NW, _NCHUNK, _CHUNK)
    y1, y2 = _sc_gather(p, idx1r, idx2r)
    y1p = y1.reshape(_BATCH * _PAD // 128, 128)
    y2p = y2.reshape(_BATCH * _PAD // 128, 128)
    loss = pl.pallas_call(
        _finish_body,
        out_shape=jax.ShapeDtypeStruct((1, _OUT), jnp.float32),
    )(y1p, y2p)
    x1 = y1[:, :_OUT][None]
    x2 = y2[:, :_OUT][None]
    return loss, x1, x2


# speedup vs baseline: 1.0932x; 1.0932x over previous
import functools

import jax
import jax.numpy as jnp
from jax import lax
from jax.experimental import pallas as pl
from jax.experimental.pallas import tpu as pltpu
from jax.experimental.pallas import tpu_sc as plsc

_VOCAB = 1000
_EMBED = 20
_OUT = 10
_BATCH = 16384
_PAD = 16
_NC, _NS = 2, 16
_NW = _NC * _NS
_PER_W = _BATCH // _NW
_CHUNK = 128
_NCHUNK = _PER_W // _CHUNK


def _build_table_body(t_hbm, w_hbm, b_hbm, o_ref, t_v, w_v, b_v, sem):
    ct = pltpu.make_async_copy(t_hbm, t_v, sem)
    cw = pltpu.make_async_copy(w_hbm, w_v, sem)
    cb = pltpu.make_async_copy(b_hbm, b_v, sem)
    ct.start(); cw.start(); cb.start()
    ct.wait(); cw.wait(); cb.wait()
    w = w_v[...]
    wt = jnp.concatenate(
        [w.T, jnp.zeros((_EMBED, _PAD - _OUT), jnp.float32)], axis=1)
    bp = jnp.concatenate(
        [b_v[...], jnp.zeros((1, _PAD - _OUT), jnp.float32)], axis=1)
    p = lax.dot_general(t_v[...], wt, (((1,), (0,)), ((), ())),
                        precision=lax.Precision.HIGHEST,
                        preferred_element_type=jnp.float32)
    o_ref[...] = p + bp


def _gather_body(p_hbm, i1_hbm, i2_hbm, y1_hbm, y2_hbm,
                 i1_v, i2_v, r1_v, r2_v, sem, osem):
    wid = lax.axis_index("s") * _NC + lax.axis_index("c")
    base = wid * _PER_W
    pltpu.sync_copy(i1_hbm.at[wid], i1_v)
    pltpu.sync_copy(i2_hbm.at[wid], i2_v)
    copies = []
    for j in range(_NCHUNK):
        copies.append(pltpu.async_copy(
            p_hbm.at[i1_v.at[j]],
            r1_v.at[pl.ds(j * _CHUNK, _CHUNK)], sem))
        copies.append(pltpu.async_copy(
            p_hbm.at[i2_v.at[j]],
            r2_v.at[pl.ds(j * _CHUNK, _CHUNK)], sem))
    for c in copies:
        c.wait()
    out1 = pltpu.async_copy(r1_v, y1_hbm.at[pl.ds(base, _PER_W)], osem)
    out2 = pltpu.async_copy(r2_v, y2_hbm.at[pl.ds(base, _PER_W)], osem)
    out1.wait()
    out2.wait()


def _sc_gather(p, idx1r, idx2r):
    mesh = plsc.VectorSubcoreMesh(core_axis_name="c", subcore_axis_name="s")
    run = pl.kernel(
        _gather_body,
        out_type=(
            jax.ShapeDtypeStruct((_BATCH, _PAD), jnp.float32),
            jax.ShapeDtypeStruct((_BATCH, _PAD), jnp.float32),
        ),
        mesh=mesh,
        compiler_params=pltpu.CompilerParams(use_tc_tiling_on_sc=False),
        scratch_types=[
            pltpu.VMEM((_NCHUNK, _CHUNK), jnp.int32),
            pltpu.VMEM((_NCHUNK, _CHUNK), jnp.int32),
            pltpu.VMEM((_PER_W, _PAD), jnp.float32),
            pltpu.VMEM((_PER_W, _PAD), jnp.float32),
            pltpu.SemaphoreType.DMA,
            pltpu.SemaphoreType.DMA,
        ],
    )
    return run(p, idx1r, idx2r)


def _finish_body(y1_ref, y2_ref, loss_ref):
    ap = y1_ref[...]
    cp = y2_ref[...]
    num128 = jnp.sum(ap * cp, axis=0)[None, :]
    s1_128 = jnp.sum(ap * ap, axis=0)[None, :]
    s2_128 = jnp.sum(cp * cp, axis=0)[None, :]
    num = jnp.zeros((1, _PAD), jnp.float32)
    s1 = jnp.zeros((1, _PAD), jnp.float32)
    s2 = jnp.zeros((1, _PAD), jnp.float32)
    for j in range(8):
        num = num + num128[:, j * _PAD:(j + 1) * _PAD]
        s1 = s1 + s1_128[:, j * _PAD:(j + 1) * _PAD]
        s2 = s2 + s2_128[:, j * _PAD:(j + 1) * _PAD]
    denom = jnp.maximum(jnp.sqrt(s1) * jnp.sqrt(s2), 1e-8)
    loss_ref[...] = (num / denom)[:, :_OUT]


def kernel(DPTD_name_1, DPTD_name_2, table, W, b):
    p = pl.pallas_call(
        _build_table_body,
        in_specs=[
            pl.BlockSpec(memory_space=pl.ANY),
            pl.BlockSpec(memory_space=pl.ANY),
            pl.BlockSpec(memory_space=pl.ANY),
        ],
        out_shape=jax.ShapeDtypeStruct((_VOCAB, _PAD), jnp.float32),
        scratch_shapes=[
            pltpu.VMEM((_VOCAB, _EMBED), jnp.float32),
            pltpu.VMEM((_OUT, _EMBED), jnp.float32),
            pltpu.VMEM((1, _OUT), jnp.float32),
            pltpu.SemaphoreType.DMA,
        ],
    )(table, W, b.reshape(1, _OUT))
    idx1r = DPTD_name_1.astype(jnp.int32).reshape(_NW, _NCHUNK, _CHUNK)
    idx2r = DPTD_name_2.astype(jnp.int32).reshape(_NW, _NCHUNK, _CHUNK)
    y1, y2 = _sc_gather(p, idx1r, idx2r)
    y1p = y1.reshape(_BATCH * _PAD // 128, 128)
    y2p = y2.reshape(_BATCH * _PAD // 128, 128)
    loss = pl.pallas_call(
        _finish_body,
        out_shape=jax.ShapeDtypeStruct((1, _OUT), jnp.float32),
    )(y1p, y2p)
    x1 = y1[:, :_OUT][None]
    x2 = y2[:, :_OUT][None]
    return loss, x1, x2


# speedup vs baseline: 2.0117x; 1.8402x over previous
import functools

import jax
import jax.numpy as jnp
from jax import lax
from jax.experimental import pallas as pl
from jax.experimental.pallas import tpu as pltpu
from jax.experimental.pallas import tpu_sc as plsc

_VOCAB = 1000
_EMBED = 20
_OUT = 10
_BATCH = 16384
_PAD = 16
_NC, _NS = 2, 16
_NW = _NC * _NS
_PER_W = _BATCH // _NW
_CHUNK = 128
_NCHUNK = _PER_W // _CHUNK


def _build_table_body(t_hbm, w_hbm, b_hbm, o_ref, t_v, w_v, b_v, sem):
    ct = pltpu.make_async_copy(t_hbm, t_v, sem)
    cw = pltpu.make_async_copy(w_hbm, w_v, sem)
    cb = pltpu.make_async_copy(b_hbm, b_v, sem)
    ct.start(); cw.start(); cb.start()
    ct.wait(); cw.wait(); cb.wait()
    w = w_v[...]
    wt = jnp.concatenate(
        [w.T, jnp.zeros((_EMBED, _PAD - _OUT), jnp.float32)], axis=1)
    bp = jnp.concatenate(
        [b_v[...], jnp.zeros((1, _PAD - _OUT), jnp.float32)], axis=1)
    p = lax.dot_general(t_v[...], wt, (((1,), (0,)), ((), ())),
                        precision=lax.Precision.HIGHEST,
                        preferred_element_type=jnp.float32)
    o_ref[...] = p + bp


def _gather_body(p_hbm, i1_hbm, i2_hbm, y1_hbm, y2_hbm,
                 i1_v, i2_v, r1_v, r2_v, sem, osem):
    wid = lax.axis_index("s") * _NC + lax.axis_index("c")
    base = wid * _PER_W
    pltpu.sync_copy(i1_hbm.at[wid], i1_v)
    pltpu.sync_copy(i2_hbm.at[wid], i2_v)
    copies = []
    for j in range(_NCHUNK):
        copies.append(pltpu.async_copy(
            p_hbm.at[i1_v.at[j]],
            r1_v.at[pl.ds(j * _CHUNK, _CHUNK)], sem))
        copies.append(pltpu.async_copy(
            p_hbm.at[i2_v.at[j]],
            r2_v.at[pl.ds(j * _CHUNK, _CHUNK)], sem))
    for c in copies:
        c.wait()
    out1 = pltpu.async_copy(r1_v, y1_hbm.at[pl.ds(base, _PER_W)], osem)
    out2 = pltpu.async_copy(r2_v, y2_hbm.at[pl.ds(base, _PER_W)], osem)
    out1.wait()
    out2.wait()


def _sc_gather(p, idx1r, idx2r):
    mesh = plsc.VectorSubcoreMesh(core_axis_name="c", subcore_axis_name="s")
    run = pl.kernel(
        _gather_body,
        out_type=(
            jax.ShapeDtypeStruct((_BATCH, _PAD), jnp.float32),
            jax.ShapeDtypeStruct((_BATCH, _PAD), jnp.float32),
        ),
        mesh=mesh,
        compiler_params=pltpu.CompilerParams(use_tc_tiling_on_sc=False),
        scratch_types=[
            pltpu.VMEM((_NCHUNK, _CHUNK), jnp.int32),
            pltpu.VMEM((_NCHUNK, _CHUNK), jnp.int32),
            pltpu.VMEM((_PER_W, _PAD), jnp.float32),
            pltpu.VMEM((_PER_W, _PAD), jnp.float32),
            pltpu.SemaphoreType.DMA,
            pltpu.SemaphoreType.DMA,
        ],
    )
    return run(p, idx1r, idx2r)


def _repack_transposed(ap, o_ref):
    a2 = ap.reshape(128, 16, 128).transpose(1, 0, 2).reshape(2048, 128)
    a4 = a2.T
    a7 = a4.reshape(8, 16, 16, 128).transpose(1, 2, 0, 3)
    a9 = a7.reshape(2048, 128).T
    for f in range(_OUT):
        o_ref[pl.ds(128 * f, 128), :] = a9[:, 128 * f:128 * f + 128]


def _finish_body(y1_ref, y2_ref, loss_ref, x1_ref, x2_ref):
    ap = y1_ref[...]
    cp = y2_ref[...]
    _repack_transposed(ap, x1_ref)
    _repack_transposed(cp, x2_ref)
    num128 = jnp.sum(ap * cp, axis=0)[None, :]
    s1_128 = jnp.sum(ap * ap, axis=0)[None, :]
    s2_128 = jnp.sum(cp * cp, axis=0)[None, :]
    num = jnp.zeros((1, _PAD), jnp.float32)
    s1 = jnp.zeros((1, _PAD), jnp.float32)
    s2 = jnp.zeros((1, _PAD), jnp.float32)
    for j in range(8):
        num = num + num128[:, j * _PAD:(j + 1) * _PAD]
        s1 = s1 + s1_128[:, j * _PAD:(j + 1) * _PAD]
        s2 = s2 + s2_128[:, j * _PAD:(j + 1) * _PAD]
    denom = jnp.maximum(jnp.sqrt(s1) * jnp.sqrt(s2), 1e-8)
    loss_ref[...] = (num / denom)[:, :_OUT]


def kernel(DPTD_name_1, DPTD_name_2, table, W, b):
    p = pl.pallas_call(
        _build_table_body,
        in_specs=[
            pl.BlockSpec(memory_space=pl.ANY),
            pl.BlockSpec(memory_space=pl.ANY),
            pl.BlockSpec(memory_space=pl.ANY),
        ],
        out_shape=jax.ShapeDtypeStruct((_VOCAB, _PAD), jnp.float32),
        scratch_shapes=[
            pltpu.VMEM((_VOCAB, _EMBED), jnp.float32),
            pltpu.VMEM((_OUT, _EMBED), jnp.float32),
            pltpu.VMEM((1, _OUT), jnp.float32),
            pltpu.SemaphoreType.DMA,
        ],
    )(table, W, b.reshape(1, _OUT))
    idx1r = DPTD_name_1.astype(jnp.int32).reshape(_NW, _NCHUNK, _CHUNK)
    idx2r = DPTD_name_2.astype(jnp.int32).reshape(_NW, _NCHUNK, _CHUNK)
    y1, y2 = _sc_gather(p, idx1r, idx2r)
    y1p = y1.reshape(_BATCH * _PAD // 128, 128)
    y2p = y2.reshape(_BATCH * _PAD // 128, 128)
    loss, x1t, x2t = pl.pallas_call(
        _finish_body,
        out_shape=(
            jax.ShapeDtypeStruct((1, _OUT), jnp.float32),
            jax.ShapeDtypeStruct((_OUT * _BATCH // 128, 128), jnp.float32),
            jax.ShapeDtypeStruct((_OUT * _BATCH // 128, 128), jnp.float32),
        ),
    )(y1p, y2p)
    x1 = x1t.reshape(_OUT, _BATCH).T[None]
    x2 = x2t.reshape(_OUT, _BATCH).T[None]
    return loss, x1, x2
